# baseline (speedup 1.0000x reference)
import jax
import jax.numpy as jnp
from jax import lax
from jax.experimental import pallas as pl
from jax.experimental.pallas import tpu as pltpu
from jax.experimental.pallas import tpu_sc as plsc

D_OUT = 4096
D_IN = 4096
BLOCK = 64
N_BLOCKS = D_IN // BLOCK
LANES = 16
GROUPS = N_BLOCKS // LANES

NC = 2
NS = 16
NW = NC * NS
ROWS_PER_W = D_OUT // NW
CHUNK_ROWS = 8
N_CHUNKS = ROWS_PER_W // CHUNK_ROWS


def _sc_body(master_hbm, scale_hbm, out_hbm, buf, sbuf):
    wid = lax.axis_index("s") * NC + lax.axis_index("c")
    base = wid * ROWS_PER_W

    def chunk_body(c, carry):
        r0 = base + c * CHUNK_ROWS
        pltpu.sync_copy(master_hbm.at[pl.ds(r0, CHUNK_ROWS)], buf)
        pltpu.sync_copy(scale_hbm.at[pl.ds(r0, CHUNK_ROWS)], sbuf)

        def row_group(i, carry2):
            r = i // GROUPS
            g = i % GROUPS
            sv = sbuf[r, pl.ds(g * LANES, LANES)]
            ssafe = jnp.where(sv == 0.0, 1.0, sv)
            r75v = 7.5 / ssafe
            mv = sv * (2.0 / 15.0)
            dnums = lax.GatherDimensionNumbers(
                offset_dims=(), collapsed_slice_dims=(0,), start_index_map=(0,))
            for k in range(LANES):
                idx = jnp.full((LANES, 1), k, jnp.int32)
                r75s = lax.gather(r75v, idx, dnums, (1,),
                                  mode=lax.GatherScatterMode.PROMISE_IN_BOUNDS)
                ms = lax.gather(mv, idx, dnums, (1,),
                                mode=lax.GatherScatterMode.PROMISE_IN_BOUNDS)
                colbase = g * (LANES * BLOCK) + k * BLOCK
                for v in range(BLOCK // LANES):
                    col = colbase + v * LANES
                    x = buf[r, pl.ds(col, LANES)]
                    u = x * r75s + 8.0
                    cnt = u.astype(jnp.int32).astype(jnp.float32)
                    buf[r, pl.ds(col, LANES)] = (cnt - 7.5) * ms
            return carry2

        lax.fori_loop(0, CHUNK_ROWS * GROUPS, row_group, 0)
        pltpu.sync_copy(buf, out_hbm.at[pl.ds(r0, CHUNK_ROWS)])
        return carry

    lax.fori_loop(0, N_CHUNKS, chunk_body, 0)


def kernel(master, scale, centroids):
    del centroids
    mesh = plsc.VectorSubcoreMesh(core_axis_name="c", subcore_axis_name="s")
    k = pl.kernel(
        _sc_body,
        mesh=mesh,
        out_type=jax.ShapeDtypeStruct((D_OUT, D_IN), jnp.float32),
        scratch_types=[
            pltpu.VMEM((CHUNK_ROWS, D_IN), jnp.float32),
            pltpu.VMEM((CHUNK_ROWS, N_BLOCKS), jnp.float32),
        ],
    )
    return k(master, scale)
